# baseline (speedup 1.0000x reference)
import functools

import jax
import jax.numpy as jnp
from jax.experimental import pallas as pl
from jax.experimental.pallas import tpu as pltpu

_SMOOTH = 0.5


def _fused_kernel(
    adj_top_ref, adj_bot_ref, x_ref, wt_ref, b_ref, eye_ref, out_ref,
    sup_ref, dmr_ref, *, n, nh, bk, g,
):
    i = pl.program_id(0)

    def _stripe_step(mask_stripe):
        a_top = adj_top_ref[...]
        a_bot = adj_bot_ref[...]
        valid = n - i * bk
        lane = jax.lax.broadcasted_iota(jnp.int32, (1, bk), 1)
        if mask_stripe:
            a_top = jnp.where(lane < valid, a_top, 0.0)
            a_bot = jnp.where(lane < valid, a_bot, 0.0)
        deg = jax.lax.dot_general(
            jnp.ones((nh, 1), jnp.float32), a_top,
            (((0,), (0,)), ((), ())),
            preferred_element_type=jnp.float32,
        ) + jax.lax.dot_general(
            jnp.ones((n - nh, 1), jnp.float32), a_bot,
            (((0,), (0,)), ((), ())),
            preferred_element_type=jnp.float32,
        )
        dm = jnp.where(lane < valid, jax.lax.rsqrt(deg + 1.0), 0.0)
        dm_col = jax.lax.dot_general(
            eye_ref[...], dm, (((1,), (1,)), ((), ())),
            preferred_element_type=jnp.float32,
        )
        sup = (
            jnp.dot(x_ref[...], wt_ref[...], preferred_element_type=jnp.float32)
            + b_ref[...]
        )
        row = jax.lax.broadcasted_iota(jnp.int32, (bk, 1), 0)
        ss = jnp.where(row < valid, dm_col * sup, 0.0).astype(jnp.bfloat16)

        @pl.when(i == 0)
        def _init():
            out_ref[...] = jnp.zeros_like(out_ref)

        out_ref[0:nh, :] += jnp.dot(
            a_top.astype(jnp.bfloat16), ss, preferred_element_type=jnp.float32
        )
        out_ref[nh:n, :] += jnp.dot(
            a_bot.astype(jnp.bfloat16), ss, preferred_element_type=jnp.float32
        )
        sup_ref[pl.ds(i * bk, bk), :] = sup
        dmr_ref[pl.ds(i, 1), :] = dm

    ragged = n % bk != 0

    @pl.when(i < (g - 1 if ragged else g))
    def _full_stripes():
        _stripe_step(mask_stripe=False)

    if ragged:
        @pl.when(i == g - 1)
        def _last_stripe():
            _stripe_step(mask_stripe=True)

    @pl.when(i == g)
    def _rescale():
        for m in range(g):
            lo = m * bk
            cb = min(bk, n - lo)
            dm_b = jax.lax.dot_general(
                eye_ref[0:cb, :], dmr_ref[pl.ds(m, 1), :],
                (((1,), (1,)), ((), ())),
                preferred_element_type=jnp.float32,
            )
            s_b = sup_ref[lo:lo + cb, :]
            v = out_ref[lo:lo + cb, :]
            feat = dm_b * v + (dm_b * dm_b) * s_b
            out_ref[lo:lo + cb, :] = (
                (feat * _SMOOTH + s_b) * (1.0 / (1.0 + _SMOOTH))
            )


@jax.jit
def kernel(x, adj, W, b):
    n, d = x.shape
    bk = 512
    g = pl.cdiv(n, bk)

    wt = W.T
    b2 = b.reshape(1, d)
    eye = jnp.eye(bk, dtype=jnp.float32)
    nh = (n // 2 + 7) // 8 * 8

    out = pl.pallas_call(
        functools.partial(_fused_kernel, n=n, nh=nh, bk=bk, g=g),
        grid=(g + 1,),
        in_specs=[
            pl.BlockSpec((nh, bk), lambda i: (0, jnp.minimum(i, g - 1))),
            pl.BlockSpec(
                (n - nh, bk),
                lambda i: (nh // (n - nh), jnp.minimum(i, g - 1)),
            ),
            pl.BlockSpec((bk, d), lambda i: (jnp.minimum(i, g - 1), 0)),
            pl.BlockSpec((d, d), lambda i: (0, 0)),
            pl.BlockSpec((1, d), lambda i: (0, 0)),
            pl.BlockSpec((bk, bk), lambda i: (0, 0)),
        ],
        out_specs=pl.BlockSpec((n, d), lambda i: (0, 0)),
        out_shape=jax.ShapeDtypeStruct((n, d), jnp.float32),
        scratch_shapes=[
            pltpu.VMEM((g * bk, d), jnp.float32),
            pltpu.VMEM((g, bk), jnp.float32),
        ],
        compiler_params=pltpu.CompilerParams(
            vmem_limit_bytes=100 * 1024 * 1024,
        ),
    )(adj, adj, x, wt, b2, eye)

    return out


# speedup vs baseline: 1.0143x; 1.0143x over previous
import functools

import jax
import jax.numpy as jnp
from jax.experimental import pallas as pl
from jax.experimental.pallas import tpu as pltpu

_SMOOTH = 0.5


def _fused_kernel(
    adj_ref, x_ref, wt_ref, b_ref, eye_ref, out_ref,
    sup_ref, dmr_ref, *, n, bk, g,
):
    i = pl.program_id(0)

    def _stripe_step(mask_stripe):
        a = adj_ref[...]
        valid = n - i * bk
        lane = jax.lax.broadcasted_iota(jnp.int32, (1, bk), 1)
        if mask_stripe:
            a = jnp.where(lane < valid, a, 0.0)
        ones = jnp.ones((n, 1), jnp.float32)
        deg = jax.lax.dot_general(
            ones, a, (((0,), (0,)), ((), ())),
            preferred_element_type=jnp.float32,
        )
        dm = jnp.where(lane < valid, jax.lax.rsqrt(deg + 1.0), 0.0)
        dm_col = jax.lax.dot_general(
            eye_ref[...], dm, (((1,), (1,)), ((), ())),
            preferred_element_type=jnp.float32,
        )
        sup = (
            jnp.dot(x_ref[...], wt_ref[...], preferred_element_type=jnp.float32)
            + b_ref[...]
        )
        row = jax.lax.broadcasted_iota(jnp.int32, (bk, 1), 0)
        ss = jnp.where(row < valid, dm_col * sup, 0.0).astype(jnp.bfloat16)

        @pl.when(i == 0)
        def _init():
            out_ref[...] = jnp.zeros_like(out_ref)

        out_ref[...] += jnp.dot(
            a.astype(jnp.bfloat16), ss, preferred_element_type=jnp.float32
        )
        sup_ref[pl.ds(i * bk, bk), :] = sup
        dmr_ref[pl.ds(i, 1), :] = dm

    ragged = n % bk != 0

    @pl.when(i < (g - 1 if ragged else g))
    def _full_stripes():
        _stripe_step(mask_stripe=False)

    if ragged:
        @pl.when(i == g - 1)
        def _last_stripe():
            _stripe_step(mask_stripe=True)

    @pl.when(i == g)
    def _rescale():
        for m in range(g):
            lo = m * bk
            cb = min(bk, n - lo)
            dm_b = jax.lax.dot_general(
                eye_ref[0:cb, :], dmr_ref[pl.ds(m, 1), :],
                (((1,), (1,)), ((), ())),
                preferred_element_type=jnp.float32,
            )
            s_b = sup_ref[lo:lo + cb, :]
            v = out_ref[lo:lo + cb, :]
            feat = dm_b * v + (dm_b * dm_b) * s_b
            out_ref[lo:lo + cb, :] = (
                (feat * _SMOOTH + s_b) * (1.0 / (1.0 + _SMOOTH))
            )


@jax.jit
def kernel(x, adj, W, b):
    n, d = x.shape
    bk = 512
    g = pl.cdiv(n, bk)

    wt = W.T
    b2 = b.reshape(1, d)
    eye = jnp.eye(bk, dtype=jnp.float32)

    out = pl.pallas_call(
        functools.partial(_fused_kernel, n=n, bk=bk, g=g),
        grid=(g + 1,),
        in_specs=[
            pl.BlockSpec((n, bk), lambda i: (0, jnp.minimum(i, g - 1))),
            pl.BlockSpec((bk, d), lambda i: (jnp.minimum(i, g - 1), 0)),
            pl.BlockSpec((d, d), lambda i: (0, 0)),
            pl.BlockSpec((1, d), lambda i: (0, 0)),
            pl.BlockSpec((bk, bk), lambda i: (0, 0)),
        ],
        out_specs=pl.BlockSpec((n, d), lambda i: (0, 0)),
        out_shape=jax.ShapeDtypeStruct((n, d), jnp.float32),
        scratch_shapes=[
            pltpu.VMEM((g * bk, d), jnp.float32),
            pltpu.VMEM((g, bk), jnp.float32),
        ],
        compiler_params=pltpu.CompilerParams(
            vmem_limit_bytes=100 * 1024 * 1024,
        ),
    )(adj, x, wt, b2, eye)

    return out
